# Initial kernel scaffold; baseline (speedup 1.0000x reference)
import jax
import jax.numpy as jnp
from jax.experimental import pallas as pl


def kernel(batch, user_table, item_table):
    raise NotImplementedError("write your pallas kernel here")


# speedup vs baseline: 10.5964x; 10.5964x over previous
import functools

import jax
import jax.numpy as jnp
from jax import lax
from jax.experimental import pallas as pl
from jax.experimental.pallas import tpu as pltpu
from jax.experimental.pallas import tpu_sc as plsc

B, L, N = 1024, 20, 50
D = 64

_info = plsc.get_sparse_core_info()
NC, NS = _info.num_cores, _info.num_subcores
NW = NC * NS

PAIRS = B * L
PAIRS_PER_TILE = PAIRS // NW
CHUNK_PAIRS = 16
CHUNK_ITEMS = CHUNK_PAIRS * N
NCHUNKS = PAIRS_PER_TILE // CHUNK_PAIRS
IDX_ROW = 100
IDX_ROWS_PER_CHUNK = CHUNK_ITEMS // IDX_ROW
NGROUPS = 4


def _vperm(x, idx):
    return lax.gather(
        x, idx.reshape(16, 1),
        lax.GatherDimensionNumbers(offset_dims=(), collapsed_slice_dims=(0,),
                                   start_index_map=(0,)),
        slice_sizes=(1,),
        mode=lax.GatherScatterMode.PROMISE_IN_BOUNDS)


def _body(uid_hbm, iid_hbm, user_table, item_table, out_hbm,
          uidx_v, iidx_v, urows_v, irows_v, out_v, sem):
    wid = lax.axis_index("c") * NS + lax.axis_index("s")
    lane = lax.iota(jnp.int32, 16)

    def chunk_body(c, _):
        pair_base = pl.multiple_of(wid * PAIRS_PER_TILE + c * CHUNK_PAIRS,
                                   CHUNK_PAIRS)
        item_base = pl.multiple_of(pair_base * N, CHUNK_ITEMS)
        idx_row_base = pl.multiple_of(item_base // IDX_ROW,
                                      IDX_ROWS_PER_CHUNK)

        pltpu.sync_copy(uid_hbm.at[pl.ds(pair_base, CHUNK_PAIRS)], uidx_v)
        pltpu.sync_copy(iid_hbm.at[pl.ds(idx_row_base, IDX_ROWS_PER_CHUNK)],
                        iidx_v)

        copies = [pltpu.async_copy(user_table.at[uidx_v], urows_v, sem)]
        for j in range(IDX_ROWS_PER_CHUNK):
            copies.append(pltpu.async_copy(
                item_table.at[iidx_v.at[j]],
                irows_v.at[pl.ds(j * IDX_ROW, IDX_ROW)], sem))
        for cp in copies:
            cp.wait()

        def group_body(t, _):
            p = t // NGROUPS
            g = t % NGROUPS
            goff = jnp.minimum(g * 16, N - 16)
            u0 = urows_v[p, pl.ds(0, 16)]
            u1 = urows_v[p, pl.ds(16, 16)]
            u2 = urows_v[p, pl.ds(32, 16)]
            u3 = urows_v[p, pl.ds(48, 16)]
            srow = p * N + goff
            acc = jnp.zeros((16,), jnp.float32)
            for j in range(16):
                r = srow + j
                prod = (irows_v[r, pl.ds(0, 16)] * u0
                        + irows_v[r, pl.ds(16, 16)] * u1
                        + irows_v[r, pl.ds(32, 16)] * u2
                        + irows_v[r, pl.ds(48, 16)] * u3)
                s_vec = prod
                for k in (8, 4, 2, 1):
                    s_vec = s_vec + _vperm(s_vec, lane ^ k)
                acc = jnp.where(lane == j, s_vec, acc)
            out_v[pl.ds(srow, 16)] = acc
            return _

        lax.fori_loop(0, CHUNK_PAIRS * NGROUPS, group_body, None)

        pltpu.sync_copy(out_v, out_hbm.at[pl.ds(item_base, CHUNK_ITEMS)])
        return _

    lax.fori_loop(0, NCHUNKS, chunk_body, None)


@functools.partial(jax.jit, static_argnums=())
def kernel(batch, user_table, item_table):
    user_ids = batch[:, :, 0].reshape(PAIRS).astype(jnp.int32)
    item_ids = batch[:, :, 1:].reshape(PAIRS * N // IDX_ROW,
                                       IDX_ROW).astype(jnp.int32)

    run = pl.kernel(
        _body,
        out_type=jax.ShapeDtypeStruct((PAIRS * N,), jnp.float32),
        mesh=plsc.VectorSubcoreMesh(core_axis_name="c", subcore_axis_name="s"),
        compiler_params=pltpu.CompilerParams(use_tc_tiling_on_sc=False),
        scratch_types=[
            pltpu.VMEM((CHUNK_PAIRS,), jnp.int32),
            pltpu.VMEM((IDX_ROWS_PER_CHUNK, IDX_ROW), jnp.int32),
            pltpu.VMEM((CHUNK_PAIRS, D), jnp.float32),
            pltpu.VMEM((CHUNK_ITEMS, D), jnp.float32),
            pltpu.VMEM((CHUNK_ITEMS,), jnp.float32),
            pltpu.SemaphoreType.DMA,
        ],
    )
    scores = run(user_ids, item_ids, user_table, item_table)
    return scores.reshape(B, L, N)


# speedup vs baseline: 10.9493x; 1.0333x over previous
import functools

import jax
import jax.numpy as jnp
from jax import lax
from jax.experimental import pallas as pl
from jax.experimental.pallas import tpu as pltpu
from jax.experimental.pallas import tpu_sc as plsc

B, L, N = 1024, 20, 50
D = 64

_info = plsc.get_sparse_core_info()
NC, NS = _info.num_cores, _info.num_subcores
NW = NC * NS

PAIRS = B * L
PAIRS_PER_TILE = PAIRS // NW
CHUNK_PAIRS = 16
CHUNK_ITEMS = CHUNK_PAIRS * N
NCHUNKS = PAIRS_PER_TILE // CHUNK_PAIRS
IDX_ROW = 100
IDX_ROWS_PER_CHUNK = CHUNK_ITEMS // IDX_ROW
NGROUPS = 4


def _vperm(x, idx):
    return lax.gather(
        x, idx.reshape(16, 1),
        lax.GatherDimensionNumbers(offset_dims=(), collapsed_slice_dims=(0,),
                                   start_index_map=(0,)),
        slice_sizes=(1,),
        mode=lax.GatherScatterMode.PROMISE_IN_BOUNDS)


BITREV4 = [0, 8, 4, 12, 2, 10, 6, 14, 1, 9, 5, 13, 3, 11, 7, 15]


def _body(uid_hbm, iid_hbm, user_table, item_table, out_hbm,
          uidx_v, iidx_v, urows_v, irows_v, out_v, sem):
    wid = lax.axis_index("c") * NS + lax.axis_index("s")
    lane = lax.iota(jnp.int32, 16)
    perm_idx = {s: lane ^ s for s in (8, 4, 2, 1)}
    sel_mask = {s: (lane & s) == 0 for s in (8, 4, 2, 1)}

    def chunk_body(c, _):
        pair_base = pl.multiple_of(wid * PAIRS_PER_TILE + c * CHUNK_PAIRS,
                                   CHUNK_PAIRS)
        item_base = pl.multiple_of(pair_base * N, CHUNK_ITEMS)
        idx_row_base = pl.multiple_of(item_base // IDX_ROW,
                                      IDX_ROWS_PER_CHUNK)

        pltpu.sync_copy(uid_hbm.at[pl.ds(pair_base, CHUNK_PAIRS)], uidx_v)
        pltpu.sync_copy(iid_hbm.at[pl.ds(idx_row_base, IDX_ROWS_PER_CHUNK)],
                        iidx_v)

        copies = [pltpu.async_copy(user_table.at[uidx_v], urows_v, sem)]
        for j in range(IDX_ROWS_PER_CHUNK):
            copies.append(pltpu.async_copy(
                item_table.at[iidx_v.at[j]],
                irows_v.at[pl.ds(j * IDX_ROW, IDX_ROW)], sem))
        for cp in copies:
            cp.wait()

        def group_body(t, _):
            p = t // NGROUPS
            g = t % NGROUPS
            goff = jnp.minimum(g * 16, N - 16)
            u0 = urows_v[p, pl.ds(0, 16)]
            u1 = urows_v[p, pl.ds(16, 16)]
            u2 = urows_v[p, pl.ds(32, 16)]
            u3 = urows_v[p, pl.ds(48, 16)]
            srow = p * N + goff
            vs = []
            for j in range(16):
                r = srow + BITREV4[j]
                vs.append((irows_v[r, pl.ds(0, 16)] * u0
                           + irows_v[r, pl.ds(16, 16)] * u1)
                          + (irows_v[r, pl.ds(32, 16)] * u2
                             + irows_v[r, pl.ds(48, 16)] * u3))
            for s in (8, 4, 2, 1):
                nxt = []
                for i in range(0, len(vs), 2):
                    a, b = vs[i], vs[i + 1]
                    pa = a + _vperm(a, perm_idx[s])
                    pb = b + _vperm(b, perm_idx[s])
                    nxt.append(jnp.where(sel_mask[s], pa, pb))
                vs = nxt
            out_v[pl.ds(srow, 16)] = vs[0]
            return _

        lax.fori_loop(0, CHUNK_PAIRS * NGROUPS, group_body, None)

        pltpu.sync_copy(out_v, out_hbm.at[pl.ds(item_base, CHUNK_ITEMS)])
        return _

    lax.fori_loop(0, NCHUNKS, chunk_body, None)


@functools.partial(jax.jit, static_argnums=())
def kernel(batch, user_table, item_table):
    user_ids = batch[:, :, 0].reshape(PAIRS).astype(jnp.int32)
    item_ids = batch[:, :, 1:].reshape(PAIRS * N // IDX_ROW,
                                       IDX_ROW).astype(jnp.int32)

    run = pl.kernel(
        _body,
        out_type=jax.ShapeDtypeStruct((PAIRS * N,), jnp.float32),
        mesh=plsc.VectorSubcoreMesh(core_axis_name="c", subcore_axis_name="s"),
        compiler_params=pltpu.CompilerParams(use_tc_tiling_on_sc=False),
        scratch_types=[
            pltpu.VMEM((CHUNK_PAIRS,), jnp.int32),
            pltpu.VMEM((IDX_ROWS_PER_CHUNK, IDX_ROW), jnp.int32),
            pltpu.VMEM((CHUNK_PAIRS, D), jnp.float32),
            pltpu.VMEM((CHUNK_ITEMS, D), jnp.float32),
            pltpu.VMEM((CHUNK_ITEMS,), jnp.float32),
            pltpu.SemaphoreType.DMA,
        ],
    )
    scores = run(user_ids, item_ids, user_table, item_table)
    return scores.reshape(B, L, N)
